# Initial kernel scaffold; baseline (speedup 1.0000x reference)
import jax
import jax.numpy as jnp
from jax.experimental import pallas as pl


def kernel(xu, xp, edge_index, Wu, bu, Wi, bi, Wl0, bl0, Wr0, Wl1, bl1, Wr1, Wc1, bc1, Wc2, bc2, Wct, bct, Wtr, btr, Woc, boc, Wot, bot):
    raise NotImplementedError("write your pallas kernel here")


# speedup vs baseline: 4.3952x; 4.3952x over previous
import functools

import jax
import jax.numpy as jnp
from jax import lax
from jax.experimental import pallas as pl
from jax.experimental.pallas import tpu as pltpu
from jax.experimental.pallas import tpu_sc as plsc

NU = 8000
NP = 2000
N = NU + NP
E = 320000
F = 128
H = 128

NC = 2
NS = 16
NW = NC * NS
K = 128
J = -(-E // (NW * K))
EPAD = NW * K * J
RPT = 624
TAILR = N - NS * RPT
NTRASH = 16
CROWS = N + NTRASH
ZR = 48

_f32 = jnp.float32
_sds = jax.ShapeDtypeStruct


@functools.partial(
    pl.kernel,
    out_type=[
        _sds((NC, N, H), _f32),
    ],
    mesh=plsc.VectorSubcoreMesh(core_axis_name="c", subcore_axis_name="s"),
    scratch_types=[
        pltpu.VMEM((J, K), jnp.int32),
        pltpu.VMEM((J, K), jnp.int32),
        pltpu.VMEM((K, H), _f32),
        pltpu.VMEM((ZR, H), _f32),
        pltpu.VMEM_SHARED((N, H), _f32),
        pltpu.SemaphoreType.DMA,
    ],
)
def _sc_agg(y_hbm, src_hbm, dsts_hbm, out_s,
            src_v, dsts_v, rows_v, z128_v, sh_s, sem):
    c = lax.axis_index("c")
    s = lax.axis_index("s")
    wid = s * NC + c

    zv = jnp.zeros((16,), _f32)

    def _fill_z128(i, _):
        z128_v[i // 8, pl.ds((i % 8) * 16, 16)] = zv
        return 0
    lax.fori_loop(0, ZR * (H // 16), _fill_z128, 0)

    for kk in range(RPT // ZR):
        pltpu.sync_copy(z128_v, sh_s.at[pl.ds(s * RPT + kk * ZR, ZR)])

    @pl.when(s == 0)
    def _zero_tail():
        pltpu.sync_copy(z128_v.at[pl.ds(0, TAILR)], sh_s.at[pl.ds(NS * RPT, TAILR)])

    pltpu.sync_copy(src_hbm.at[wid], src_v)
    pltpu.sync_copy(dsts_hbm.at[wid], dsts_v)

    plsc.subcore_barrier()

    def _edge_chunk(j, _):
        pltpu.async_copy(y_hbm.at[src_v.at[j]], rows_v, sem).wait()
        pltpu.sync_copy(rows_v, sh_s.at[dsts_v.at[j]], add=True)
        return 0
    lax.fori_loop(0, J, _edge_chunk, 0)

    plsc.subcore_barrier()

    pltpu.sync_copy(sh_s.at[pl.ds(s * RPT, RPT)], out_s.at[c, pl.ds(s * RPT, RPT)])

    @pl.when(s == 0)
    def _copy_tail():
        pltpu.sync_copy(sh_s.at[pl.ds(NS * RPT, TAILR)],
                        out_s.at[c, pl.ds(NS * RPT, TAILR)])


@functools.partial(
    pl.kernel,
    out_type=[
        _sds((NC, N, 128), _f32),
    ],
    mesh=plsc.VectorSubcoreMesh(core_axis_name="c", subcore_axis_name="s"),
    scratch_types=[
        pltpu.VMEM((J, K), jnp.int32),
        pltpu.VMEM((K, 128), _f32),
        pltpu.VMEM((ZR, 128), _f32),
        pltpu.VMEM_SHARED((CROWS, 128), _f32),
    ],
)
def _sc_count(dstc_hbm, out_c, dstc_v, ones_v, z16_v, sh_c):
    c = lax.axis_index("c")
    s = lax.axis_index("s")
    wid = s * NC + c

    zv = jnp.zeros((16,), _f32)
    ov = jnp.ones((16,), _f32)

    def _fill_z16(i, _):
        z16_v[i // 8, pl.ds((i % 8) * 16, 16)] = zv
        return 0
    lax.fori_loop(0, ZR * 8, _fill_z16, 0)

    def _fill_ones(i, _):
        ones_v[i // 8, pl.ds((i % 8) * 16, 16)] = ov
        return 0
    lax.fori_loop(0, K * 8, _fill_ones, 0)

    for kk in range(RPT // ZR):
        pltpu.sync_copy(z16_v, sh_c.at[pl.ds(s * RPT + kk * ZR, ZR)])

    @pl.when(s == 0)
    def _zero_tail():
        pltpu.sync_copy(z16_v.at[pl.ds(0, CROWS - NS * RPT)],
                        sh_c.at[pl.ds(NS * RPT, CROWS - NS * RPT)])

    pltpu.sync_copy(dstc_hbm.at[wid], dstc_v)

    plsc.subcore_barrier()

    def _edge_chunk(j, _):
        pltpu.sync_copy(ones_v, sh_c.at[dstc_v.at[j]], add=True)
        return 0
    lax.fori_loop(0, J, _edge_chunk, 0)

    plsc.subcore_barrier()

    pltpu.sync_copy(sh_c.at[pl.ds(s * RPT, RPT)], out_c.at[c, pl.ds(s * RPT, RPT)])

    @pl.when(s == 0)
    def _copy_tail():
        pltpu.sync_copy(sh_c.at[pl.ds(NS * RPT, TAILR)],
                        out_c.at[c, pl.ds(NS * RPT, TAILR)])


_BR0 = 2000
_BR2 = 2000

_full = lambda shape: pl.BlockSpec(shape, lambda i: tuple(0 for _ in shape))


def _row_spec(w):
    return pl.BlockSpec((_BR0, w), lambda i: (i, 0))


def _tc_stage0(xall, Wu, bu, Wi, bi, Wl0, Wr0, bl0):
    def body(x_ref, wu_ref, bu_ref, wi_ref, bi_ref, wl_ref, wr_ref, bl_ref,
             emb_ref, y_ref, z_ref):
        i = pl.program_id(0)
        W = jnp.where(i < NU // _BR0, wu_ref[...], wi_ref[...])
        b = jnp.where(i < NU // _BR0, bu_ref[...], bi_ref[...])
        h = jnp.dot(x_ref[...], W, preferred_element_type=_f32) + b
        emb_ref[...] = h
        y_ref[...] = jnp.dot(h, wl_ref[...], preferred_element_type=_f32)
        z_ref[...] = jnp.dot(h, wr_ref[...], preferred_element_type=_f32) + bl_ref[...]

    return pl.pallas_call(
        body,
        grid=(N // _BR0,),
        in_specs=[_row_spec(F), _full((F, H)), _full((1, H)), _full((F, H)),
                  _full((1, H)), _full((H, H)), _full((H, H)), _full((1, H))],
        out_specs=[_row_spec(H)] * 3,
        out_shape=[_sds((N, H), _f32)] * 3,
    )(xall, Wu, bu, Wi, bi, Wl0, Wr0, bl0)


def _mean_from_partials(s_ref, c_ref, z_ref):
    sv = s_ref[...]
    cv = c_ref[...]
    ssum = sv[0] + sv[1]
    cnt = cv[0, :, 0:1] + cv[1, :, 0:1]
    invc = 1.0 / jnp.maximum(cnt, 1.0)
    return jnp.maximum(ssum * invc + z_ref[...], 0.0)


def _tc_stage1(S0, C0, Z0, Wl1, Wr1, bl1):
    def body(s_ref, c_ref, z_ref, wl_ref, wr_ref, bl_ref,
             emb1_ref, y_ref, zo_ref):
        e1 = _mean_from_partials(s_ref, c_ref, z_ref)
        emb1_ref[...] = e1
        y_ref[...] = jnp.dot(e1, wl_ref[...], preferred_element_type=_f32)
        zo_ref[...] = jnp.dot(e1, wr_ref[...], preferred_element_type=_f32) + bl_ref[...]

    pspec = pl.BlockSpec((NC, _BR0, H), lambda i: (0, i, 0))
    cspec = pl.BlockSpec((NC, _BR0, 128), lambda i: (0, i, 0))
    return pl.pallas_call(
        body,
        grid=(N // _BR0,),
        in_specs=[pspec, cspec, _row_spec(H), _full((H, H)), _full((H, H)),
                  _full((1, H))],
        out_specs=[_row_spec(H)] * 3,
        out_shape=[_sds((N, H), _f32)] * 3,
    )(S0, C0, Z0, Wl1, Wr1, bl1)


def _tc_stage2(S1, C0, Z1, emb, emb1, Wc1a, Wc1b, Wc1c, bc1, Wc2, bc2,
               Wct, bct, Wtr, btr, Woc, boc, Wot, bot):
    def body(s_ref, c_ref, z_ref, e0_ref, e1_ref, wa_ref, wb_ref, wc_ref,
             b1_ref, w2_ref, b2_ref, wct_ref, bct_ref, wtr_ref, btr_ref,
             woc_ref, boc_ref, wot_ref, bot_ref,
             o1_ref, o0_ref, h1_ref, h0_ref):
        e2 = _mean_from_partials(s_ref, c_ref, z_ref)
        dot = lambda a, b: jnp.dot(a, b, preferred_element_type=_f32)
        hid = jnp.maximum(
            dot(e0_ref[...], wa_ref[...]) + dot(e1_ref[...], wb_ref[...])
            + dot(e2, wc_ref[...]) + b1_ref[...], 0.0)
        hid = jnp.maximum(dot(hid, w2_ref[...]) + b2_ref[...], 0.0)
        h0 = jnp.maximum(dot(hid, wct_ref[...]) + bct_ref[...], 0.0)
        h1 = jnp.maximum(dot(hid, wtr_ref[...]) + btr_ref[...], 0.0)
        h0_ref[...] = h0
        h1_ref[...] = h1
        o0_ref[...] = jnp.maximum(dot(h0, woc_ref[...]) + boc_ref[...], 0.0)
        o1_ref[...] = jnp.maximum(dot(h1, wot_ref[...]) + bot_ref[...], 0.0)

    rs = lambda w: pl.BlockSpec((_BR2, w), lambda i: (i, 0))
    pspec = pl.BlockSpec((NC, _BR2, H), lambda i: (0, i, 0))
    cspec = pl.BlockSpec((NC, _BR2, 128), lambda i: (0, i, 0))
    return pl.pallas_call(
        body,
        grid=(NU // _BR2,),
        in_specs=[pspec, cspec, rs(H), rs(H), rs(H),
                  _full((H, H)), _full((H, H)), _full((H, H)), _full((1, H)),
                  _full((H, H)), _full((1, H)),
                  _full((H, H // 2)), _full((1, H // 2)),
                  _full((H, H // 2)), _full((1, H // 2)),
                  _full((H // 2, 1)), _full((1, 1)),
                  _full((H // 2, 1)), _full((1, 1))],
        out_specs=[rs(1), rs(1), rs(H // 2), rs(H // 2)],
        out_shape=[_sds((NU, 1), _f32), _sds((NU, 1), _f32),
                   _sds((NU, H // 2), _f32), _sds((NU, H // 2), _f32)],
    )(S1, C0, Z1, emb, emb1, Wc1a, Wc1b, Wc1c, bc1, Wc2, bc2,
      Wct, bct, Wtr, btr, Woc, boc, Wot, bot)


def kernel(xu, xp, edge_index, Wu, bu, Wi, bi, Wl0, bl0, Wr0, Wl1, bl1, Wr1,
           Wc1, bc1, Wc2, bc2, Wct, bct, Wtr, btr, Woc, boc, Wot, bot):
    src = edge_index[0].astype(jnp.int32)
    dst = edge_index[1].astype(jnp.int32)

    pad = EPAD - E
    src3 = jnp.concatenate([src, jnp.full((pad,), N, jnp.int32)]).reshape(NW, J, K)
    dsts3 = jnp.concatenate([dst, jnp.zeros((pad,), jnp.int32)]).reshape(NW, J, K)
    dstc3 = jnp.concatenate([dst, jnp.full((pad,), N + 1, jnp.int32)]).reshape(NW, J, K)

    r1 = lambda v: v.reshape(1, -1)
    xall = jnp.concatenate([xu, xp], axis=0)
    zrows = jnp.zeros((8, H), _f32)

    emb, Y0, Z0 = _tc_stage0(xall, Wu, r1(bu), Wi, r1(bi), Wl0, Wr0, r1(bl0))
    [C0] = _sc_count(dstc3)
    [S0] = _sc_agg(jnp.concatenate([Y0, zrows], axis=0), src3, dsts3)
    emb1, Y1, Z1 = _tc_stage1(S0, C0, Z0, Wl1, Wr1, r1(bl1))
    [S1] = _sc_agg(jnp.concatenate([Y1, zrows], axis=0), src3, dsts3)
    o1, o0, h1, h0 = _tc_stage2(
        S1, C0, Z1, emb, emb1,
        Wc1[:H], Wc1[H:2 * H], Wc1[2 * H:], r1(bc1), Wc2, r1(bc2),
        Wct, r1(bct), Wtr, r1(btr), Woc, r1(boc), Wot, r1(bot))
    return (o1, o0, h1, h0)
